# baseline (speedup 1.0000x reference)
import jax
import jax.numpy as jnp
from jax import lax
from jax.experimental import pallas as pl
from jax.experimental.pallas import tpu as pltpu
from jax.experimental.pallas import tpu_sc as plsc

BS = 2
C = 96
H = W = 384
NPIX = H * W
ROWS = H + 3
STRIDE = 392
ACC_PER_TILE = 9808
ACC_WORDS = 16 * ACC_PER_TILE
PX_PER_TILE = NPIX // 16
GROUPS = PX_PER_TILE // 16
TAP_ROWS = GROUPS // 8
OUT_ROW_WORDS = 24 * STRIDE
CLIP_HI = 385.0


def _sc_body(g_hbm, out_hbm, gi_v, gj_v,
             i00, i01, i10, i11, w00, w01, w10, w11,
             zbuf, acc):
    c = lax.axis_index("c")
    s = lax.axis_index("s")

    gbase = c * (2 * NPIX) + s * PX_PER_TILE
    pltpu.sync_copy(g_hbm.at[pl.ds(gbase, PX_PER_TILE)], gi_v)
    pltpu.sync_copy(g_hbm.at[pl.ds(gbase + NPIX, PX_PER_TILE)], gj_v)

    def _zero(k, _):
        zbuf[pl.ds(k * 16, 16)] = jnp.zeros((16,), jnp.float32)
        return _
    lax.fori_loop(0, ACC_PER_TILE // 16, _zero, None)
    pltpu.sync_copy(zbuf.at[pl.ds(0, ACC_PER_TILE)],
                    acc.at[pl.ds(s * ACC_PER_TILE, ACC_PER_TILE)])

    def _compute(g, _):
        col = g * 16
        gi = gi_v[pl.ds(g * 16, 16)]
        gj = gj_v[pl.ds(g * 16, 16)]
        ti = jnp.minimum(jnp.maximum(gi * 192.0 + 193.0, 0.0), CLIP_HI)
        tj = jnp.minimum(jnp.maximum(gj * 192.0 + 193.0, 0.0), CLIP_HI)
        li = ti.astype(jnp.int32)
        lj = tj.astype(jnp.int32)
        fi = ti - li.astype(jnp.float32)
        fj = tj - lj.astype(jnp.float32)
        base = li * STRIDE + lj
        i00[pl.ds(col, 16)] = base
        i01[pl.ds(col, 16)] = base + 1
        i10[pl.ds(col, 16)] = base + STRIDE
        i11[pl.ds(col, 16)] = base + (STRIDE + 1)
        a0 = 1.0 - fi
        b0 = 1.0 - fj
        w00[pl.ds(col, 16)] = a0 * b0
        w01[pl.ds(col, 16)] = a0 * fj
        w10[pl.ds(col, 16)] = fi * b0
        w11[pl.ds(col, 16)] = fi * fj
        return _
    lax.fori_loop(0, GROUPS, _compute, None)

    plsc.subcore_barrier()

    pltpu.sync_copy(w00, acc.at[i00], add=True)
    pltpu.sync_copy(w01, acc.at[i01], add=True)
    pltpu.sync_copy(w10, acc.at[i10], add=True)
    pltpu.sync_copy(w11, acc.at[i11], add=True)

    plsc.subcore_barrier()

    off = (1 + s * 24) * STRIDE
    pltpu.sync_copy(acc.at[pl.ds(off, OUT_ROW_WORDS)],
                    zbuf.at[pl.ds(0, OUT_ROW_WORDS)])
    pltpu.sync_copy(zbuf.at[pl.ds(0, OUT_ROW_WORDS)],
                    out_hbm.at[pl.ds(c * (H * STRIDE) + s * OUT_ROW_WORDS,
                                     OUT_ROW_WORDS)])


_sc_scatter = pl.kernel(
    _sc_body,
    out_type=jax.ShapeDtypeStruct((BS * H * STRIDE,), jnp.float32),
    mesh=plsc.VectorSubcoreMesh(core_axis_name="c", subcore_axis_name="s"),
    scratch_types=[
        pltpu.VMEM((PX_PER_TILE,), jnp.float32),
        pltpu.VMEM((PX_PER_TILE,), jnp.float32),
        pltpu.VMEM((PX_PER_TILE,), jnp.int32),
        pltpu.VMEM((PX_PER_TILE,), jnp.int32),
        pltpu.VMEM((PX_PER_TILE,), jnp.int32),
        pltpu.VMEM((PX_PER_TILE,), jnp.int32),
        pltpu.VMEM((PX_PER_TILE,), jnp.float32),
        pltpu.VMEM((PX_PER_TILE,), jnp.float32),
        pltpu.VMEM((PX_PER_TILE,), jnp.float32),
        pltpu.VMEM((PX_PER_TILE,), jnp.float32),
        pltpu.VMEM((ACC_PER_TILE,), jnp.float32),
        pltpu.VMEM_SHARED((ACC_WORDS,), jnp.float32),
    ],
)


CB = 24


def _bcast_body(b_ref, o_ref):
    blk = b_ref[0, :, 1:W + 1]
    o_ref[0] = jnp.broadcast_to(blk[None], (CB, H, W))


def kernel(x, inv_grid):
    g = jnp.transpose(inv_grid, (0, 3, 1, 2)).reshape(BS * 2 * NPIX)
    bacc = _sc_scatter(g).reshape(BS, H, STRIDE)

    out = pl.pallas_call(
        _bcast_body,
        grid=(BS, C // CB),
        in_specs=[pl.BlockSpec((1, H, STRIDE), lambda b, k: (b, 0, 0))],
        out_specs=pl.BlockSpec((1, CB, H, W), lambda b, k: (b, k, 0, 0)),
        out_shape=jax.ShapeDtypeStruct((BS, C, H, W), x.dtype),
    )(bacc)
    return out


# speedup vs baseline: 1.0192x; 1.0192x over previous
import jax
import jax.numpy as jnp
from jax import lax
from jax.experimental import pallas as pl
from jax.experimental.pallas import tpu as pltpu
from jax.experimental.pallas import tpu_sc as plsc

BS = 2
C = 96
H = W = 384
NPIX = H * W
ROWS = H + 3
STRIDE = 392
ACC_PER_TILE = 9808
ACC_WORDS = 16 * ACC_PER_TILE
PX_PER_TILE = NPIX // 16
GROUPS = PX_PER_TILE // 16
TAP_ROWS = GROUPS // 8
OUT_ROW_WORDS = 24 * STRIDE
CLIP_HI = 385.0


def _sc_body(g_hbm, out_hbm, gi_v, gj_v,
             i00, i01, i10, i11, w00, w01, w10, w11,
             zbuf, acc):
    c = lax.axis_index("c")
    s = lax.axis_index("s")

    gbase = c * (2 * NPIX) + s * PX_PER_TILE
    pltpu.sync_copy(g_hbm.at[pl.ds(gbase, PX_PER_TILE)], gi_v)
    pltpu.sync_copy(g_hbm.at[pl.ds(gbase + NPIX, PX_PER_TILE)], gj_v)

    def _zero(k, _):
        zbuf[pl.ds(k * 16, 16)] = jnp.zeros((16,), jnp.float32)
        return _
    lax.fori_loop(0, ACC_PER_TILE // 16, _zero, None)
    pltpu.sync_copy(zbuf.at[pl.ds(0, ACC_PER_TILE)],
                    acc.at[pl.ds(s * ACC_PER_TILE, ACC_PER_TILE)])

    def _compute(g, _):
        col = g * 16
        gi = gi_v[pl.ds(g * 16, 16)]
        gj = gj_v[pl.ds(g * 16, 16)]
        ti = jnp.minimum(jnp.maximum(gi * 192.0 + 193.0, 0.0), CLIP_HI)
        tj = jnp.minimum(jnp.maximum(gj * 192.0 + 193.0, 0.0), CLIP_HI)
        li = ti.astype(jnp.int32)
        lj = tj.astype(jnp.int32)
        fi = ti - li.astype(jnp.float32)
        fj = tj - lj.astype(jnp.float32)
        base = li * STRIDE + lj
        i00[pl.ds(col, 16)] = base
        i01[pl.ds(col, 16)] = base + 1
        i10[pl.ds(col, 16)] = base + STRIDE
        i11[pl.ds(col, 16)] = base + (STRIDE + 1)
        a0 = 1.0 - fi
        b0 = 1.0 - fj
        w00[pl.ds(col, 16)] = a0 * b0
        w01[pl.ds(col, 16)] = a0 * fj
        w10[pl.ds(col, 16)] = fi * b0
        w11[pl.ds(col, 16)] = fi * fj
        return _
    lax.fori_loop(0, GROUPS, _compute, None)

    plsc.subcore_barrier()

    pltpu.sync_copy(w00, acc.at[i00], add=True)
    pltpu.sync_copy(w01, acc.at[i01], add=True)
    pltpu.sync_copy(w10, acc.at[i10], add=True)
    pltpu.sync_copy(w11, acc.at[i11], add=True)

    plsc.subcore_barrier()

    off = (1 + s * 24) * STRIDE
    pltpu.sync_copy(acc.at[pl.ds(off, OUT_ROW_WORDS)],
                    zbuf.at[pl.ds(0, OUT_ROW_WORDS)])
    pltpu.sync_copy(zbuf.at[pl.ds(0, OUT_ROW_WORDS)],
                    out_hbm.at[pl.ds(c * (H * STRIDE) + s * OUT_ROW_WORDS,
                                     OUT_ROW_WORDS)])


_sc_scatter = pl.kernel(
    _sc_body,
    out_type=jax.ShapeDtypeStruct((BS * H * STRIDE,), jnp.float32),
    mesh=plsc.VectorSubcoreMesh(core_axis_name="c", subcore_axis_name="s"),
    scratch_types=[
        pltpu.VMEM((PX_PER_TILE,), jnp.float32),
        pltpu.VMEM((PX_PER_TILE,), jnp.float32),
        pltpu.VMEM((PX_PER_TILE,), jnp.int32),
        pltpu.VMEM((PX_PER_TILE,), jnp.int32),
        pltpu.VMEM((PX_PER_TILE,), jnp.int32),
        pltpu.VMEM((PX_PER_TILE,), jnp.int32),
        pltpu.VMEM((PX_PER_TILE,), jnp.float32),
        pltpu.VMEM((PX_PER_TILE,), jnp.float32),
        pltpu.VMEM((PX_PER_TILE,), jnp.float32),
        pltpu.VMEM((PX_PER_TILE,), jnp.float32),
        pltpu.VMEM((ACC_PER_TILE,), jnp.float32),
        pltpu.VMEM_SHARED((ACC_WORDS,), jnp.float32),
    ],
)


CB = 8


def _bcast_body(b_ref, o_ref, s0, s1, sem):
    descs = []
    for b, s in ((0, s0), (1, s1)):
        s[...] = jnp.broadcast_to(b_ref[b, :, 1:W + 1][None], (CB, H, W))
        for c in range(0, C, CB):
            d = pltpu.make_async_copy(s, o_ref.at[b, pl.ds(c, CB)], sem)
            d.start()
            descs.append(d)
    for d in descs:
        d.wait()


def kernel(x, inv_grid):
    g = jnp.transpose(inv_grid, (0, 3, 1, 2)).reshape(BS * 2 * NPIX)
    bacc = _sc_scatter(g).reshape(BS, H, STRIDE)

    out = pl.pallas_call(
        _bcast_body,
        in_specs=[pl.BlockSpec(memory_space=pltpu.VMEM)],
        out_specs=pl.BlockSpec(memory_space=pltpu.MemorySpace.HBM),
        out_shape=jax.ShapeDtypeStruct((BS, C, H, W), x.dtype),
        scratch_shapes=[
            pltpu.VMEM((CB, H, W), jnp.float32),
            pltpu.VMEM((CB, H, W), jnp.float32),
            pltpu.SemaphoreType.DMA,
        ],
    )(bacc)
    return out


# speedup vs baseline: 1.0600x; 1.0400x over previous
import jax
import jax.numpy as jnp
from jax import lax
from jax.experimental import pallas as pl
from jax.experimental.pallas import tpu as pltpu
from jax.experimental.pallas import tpu_sc as plsc

BS = 2
C = 96
H = W = 384
NPIX = H * W
ROWS = H + 3
STRIDE = 392
ACC_PER_TILE = 9808
ACC_WORDS = 16 * ACC_PER_TILE
PX_PER_TILE = NPIX // 16
GROUPS = PX_PER_TILE // 16
TAP_ROWS = GROUPS // 8
OUT_ROW_WORDS = 24 * STRIDE
CLIP_HI = 385.0


NCH = 4
GR_PER_CH = GROUPS // NCH
CH_PX = GR_PER_CH * 16


def _sc_body(g_hbm, out_hbm, gv,
             i00, i01, i10, i11, w00, w01, w10, w11,
             zbuf, acc, sem):
    c = lax.axis_index("c")
    s = lax.axis_index("s")

    gbase = c * (2 * NPIX) + s * PX_PER_TILE
    ld0 = pltpu.make_async_copy(g_hbm.at[pl.ds(gbase, PX_PER_TILE)],
                                gv.at[pl.ds(0, PX_PER_TILE)], sem)
    ld1 = pltpu.make_async_copy(g_hbm.at[pl.ds(gbase + NPIX, PX_PER_TILE)],
                                gv.at[pl.ds(PX_PER_TILE, PX_PER_TILE)], sem)
    ld0.start()
    ld1.start()

    def _zero(k, _):
        zbuf[pl.ds(k * 16, 16)] = jnp.zeros((16,), jnp.float32)
        return _
    lax.fori_loop(0, ACC_PER_TILE // 16, _zero, None)
    pltpu.sync_copy(zbuf.at[pl.ds(0, ACC_PER_TILE)],
                    acc.at[pl.ds(s * ACC_PER_TILE, ACC_PER_TILE)])

    plsc.subcore_barrier()
    ld0.wait()
    ld1.wait()

    def _compute(g, _):
        col = g * 16
        gi = gv[pl.ds(g * 16, 16)]
        gj = gv[pl.ds(PX_PER_TILE + g * 16, 16)]
        ti = jnp.minimum(jnp.maximum(gi * 192.0 + 193.0, 0.0), CLIP_HI)
        tj = jnp.minimum(jnp.maximum(gj * 192.0 + 193.0, 0.0), CLIP_HI)
        li = ti.astype(jnp.int32)
        lj = tj.astype(jnp.int32)
        fi = ti - li.astype(jnp.float32)
        fj = tj - lj.astype(jnp.float32)
        base = li * STRIDE + lj
        i00[pl.ds(col, 16)] = base
        i01[pl.ds(col, 16)] = base + 1
        i10[pl.ds(col, 16)] = base + STRIDE
        i11[pl.ds(col, 16)] = base + (STRIDE + 1)
        a0 = 1.0 - fi
        b0 = 1.0 - fj
        w00[pl.ds(col, 16)] = a0 * b0
        w01[pl.ds(col, 16)] = a0 * fj
        w10[pl.ds(col, 16)] = fi * b0
        w11[pl.ds(col, 16)] = fi * fj
        return _

    descs = []
    for k in range(NCH):
        lax.fori_loop(k * GR_PER_CH, (k + 1) * GR_PER_CH, _compute, None)
        sl = pl.ds(k * CH_PX, CH_PX)
        for iv, wv in ((i00, w00), (i01, w01), (i10, w10), (i11, w11)):
            descs.append(
                pltpu.async_copy(wv.at[sl], acc.at[iv.at[sl]], sem, add=True))
    for d in descs:
        d.wait()

    plsc.subcore_barrier()

    off = (1 + s * 24) * STRIDE
    pltpu.sync_copy(acc.at[pl.ds(off, OUT_ROW_WORDS)],
                    zbuf.at[pl.ds(0, OUT_ROW_WORDS)])
    pltpu.sync_copy(zbuf.at[pl.ds(0, OUT_ROW_WORDS)],
                    out_hbm.at[pl.ds(c * (H * STRIDE) + s * OUT_ROW_WORDS,
                                     OUT_ROW_WORDS)])


_sc_scatter = pl.kernel(
    _sc_body,
    out_type=jax.ShapeDtypeStruct((BS * H * STRIDE,), jnp.float32),
    mesh=plsc.VectorSubcoreMesh(core_axis_name="c", subcore_axis_name="s"),
    scratch_types=[
        pltpu.VMEM((2 * PX_PER_TILE,), jnp.float32),
        pltpu.VMEM((PX_PER_TILE,), jnp.int32),
        pltpu.VMEM((PX_PER_TILE,), jnp.int32),
        pltpu.VMEM((PX_PER_TILE,), jnp.int32),
        pltpu.VMEM((PX_PER_TILE,), jnp.int32),
        pltpu.VMEM((PX_PER_TILE,), jnp.float32),
        pltpu.VMEM((PX_PER_TILE,), jnp.float32),
        pltpu.VMEM((PX_PER_TILE,), jnp.float32),
        pltpu.VMEM((PX_PER_TILE,), jnp.float32),
        pltpu.VMEM((ACC_PER_TILE,), jnp.float32),
        pltpu.VMEM_SHARED((ACC_WORDS,), jnp.float32),
        pltpu.SemaphoreType.DMA,
    ],
)


CB = 8


def _bcast_body(b_ref, o_ref, s0, s1, sem):
    descs = []
    for b, s in ((0, s0), (1, s1)):
        s[...] = jnp.broadcast_to(b_ref[b, :, 1:W + 1][None], (CB, H, W))
        for c in range(0, C, CB):
            d = pltpu.make_async_copy(s, o_ref.at[b, pl.ds(c, CB)], sem)
            d.start()
            descs.append(d)
    for d in descs:
        d.wait()


def kernel(x, inv_grid):
    g = jnp.transpose(inv_grid, (0, 3, 1, 2)).reshape(BS * 2 * NPIX)
    bacc = _sc_scatter(g).reshape(BS, H, STRIDE)

    out = pl.pallas_call(
        _bcast_body,
        in_specs=[pl.BlockSpec(memory_space=pltpu.VMEM)],
        out_specs=pl.BlockSpec(memory_space=pltpu.MemorySpace.HBM),
        out_shape=jax.ShapeDtypeStruct((BS, C, H, W), x.dtype),
        scratch_shapes=[
            pltpu.VMEM((CB, H, W), jnp.float32),
            pltpu.VMEM((CB, H, W), jnp.float32),
            pltpu.SemaphoreType.DMA,
        ],
    )(bacc)
    return out
